# Initial kernel scaffold; baseline (speedup 1.0000x reference)
import jax
import jax.numpy as jnp
from jax.experimental import pallas as pl


def kernel(x, pos_table):
    raise NotImplementedError("write your pallas kernel here")


# speedup vs baseline: 5.0431x; 5.0431x over previous
import jax
import jax.numpy as jnp
from jax import lax
from jax.experimental import pallas as pl
from jax.experimental.pallas import tpu as pltpu
from jax.experimental.pallas import tpu_sc as plsc

_NUM_CORES = 2
_NUM_SUBCORES = 16
_NUM_WORKERS = _NUM_CORES * _NUM_SUBCORES


def _make_sc_body(block, per_worker, rep):
    n_dma = per_worker // rep

    def body(table_hbm, out_hbm, buf, sem):
        wid = lax.axis_index("s") * _NUM_CORES + lax.axis_index("c")
        base = wid * (per_worker * block)
        for i in range(rep):
            pltpu.sync_copy(
                table_hbm.at[pl.ds(0, block)], buf.at[pl.ds(i * block, block)]
            )
        copies = []
        for j in range(n_dma):
            copies.append(
                pltpu.async_copy(
                    buf,
                    out_hbm.at[pl.ds(base + j * rep * block, rep * block)],
                    sem,
                )
            )
        for c in copies:
            c.wait()

    return body


def kernel(x, pos_table):
    N, S = x.shape
    D = pos_table.shape[1]
    block = S * D
    per_worker = N // _NUM_WORKERS
    assert per_worker * _NUM_WORKERS == N and block % 8 == 0
    rep = 8
    while per_worker % rep:
        rep //= 2

    mesh = plsc.VectorSubcoreMesh(core_axis_name="c", subcore_axis_name="s")
    k = pl.kernel(
        _make_sc_body(block, per_worker, rep),
        out_type=jax.ShapeDtypeStruct((N * block,), jnp.float32),
        mesh=mesh,
        scratch_types=[
            pltpu.VMEM((rep * block,), jnp.float32),
            pltpu.SemaphoreType.DMA,
        ],
    )
    flat = k(pos_table.reshape(-1))
    return flat.reshape(N, S, D)
